# baseline (speedup 1.0000x reference)
import functools

import jax
import jax.numpy as jnp
from jax import lax
from jax.experimental import pallas as pl
from jax.experimental.pallas import tpu as pltpu
from jax.experimental.pallas import tpu_sc as plsc

NCLS = 21
NB = 16
NPIX = 512 * 512
ROWS = NB * NPIX // 128
SROWS = NPIX // 128
WROWS = SROWS // 2
CR = 256
NCH = WROWS // CR
HWORDS = 512
PWORDS = WROWS * 32

_mesh = plsc.VectorSubcoreMesh(core_axis_name="c", subcore_axis_name="s")


@functools.partial(
    pl.kernel,
    out_type=jax.ShapeDtypeStruct((ROWS, 128), jnp.int32),
    mesh=_mesh,
    compiler_params=pltpu.CompilerParams(needs_layout_passes=False),
    scratch_types=[
        pltpu.VMEM((CR, 128), jnp.int32),
        pltpu.VMEM((CR, 128), jnp.int32),
        pltpu.VMEM((PWORDS,), jnp.int32),
        pltpu.VMEM((HWORDS,), jnp.float32),
        pltpu.VMEM((HWORDS,), jnp.float32),
        pltpu.VMEM((32,), jnp.float32),
        pltpu.VMEM((1024,), jnp.int32),
        pltpu.VMEM_SHARED((16 * HWORDS,), jnp.float32),
        pltpu.SemaphoreType.DMA,
        pltpu.SemaphoreType.DMA,
        pltpu.SemaphoreType.DMA,
        pltpu.SemaphoreType.DMA,
    ],
)
def _sc_weight_map(mask_hbm, out_hbm, buf0, buf1,
                   pbuf, hist, hist2, wtab, ptab, shist,
                   semi0, semi1, semo0, semo1):
    c = lax.axis_index("c")
    s = lax.axis_index("s")
    b = c * 8 + s // 2
    rbase = b * SROWS + (s % 2) * WROWS

    lane = lax.iota(jnp.int32, 16)
    zeros = jnp.zeros((16,), jnp.float32)
    ones = jnp.ones((16,), jnp.float32)

    bufs = [buf0, buf1]
    semis = [semi0, semi1]
    semos = [semo0, semo1]

    def row0(ci):
        return rbase + ci * CR

    for k in range(32):
        hist[pl.ds(16 * k, 16)] = zeros

    cps = [
        pltpu.async_copy(mask_hbm.at[pl.ds(row0(0), CR), :], buf0, semi0),
        pltpu.async_copy(mask_hbm.at[pl.ds(row0(1), CR), :], buf1, semi1),
    ]
    for ci in range(NCH):
        k = ci % 2
        cps[k].wait()
        buf = bufs[k]
        pbase = ci * (CR * 32)

        @plsc.parallel_loop(0, CR, step=1, unroll=2)
        def _h(r):
            for u in range(2):
                vs = []
                for t in range(4):
                    v = buf[r, pl.ds(u * 64 + t * 16, 16)]
                    plsc.addupdate_scatter(hist, [v * 16 + lane], ones)
                    vs.append(v)
                packed = (vs[0] | (vs[1] << 5)) | ((vs[2] << 10) | (vs[3] << 15))
                pbuf[pl.ds(pbase + r * 32 + u * 16, 16)] = packed

        if ci + 2 < NCH:
            cps[k] = pltpu.async_copy(
                mask_hbm.at[pl.ds(row0(ci + 2), CR), :], bufs[k], semis[k])

    pltpu.sync_copy(hist, shist.at[pl.ds(s * HWORDS, HWORDS)])
    plsc.subcore_barrier()
    pltpu.sync_copy(shist.at[pl.ds((s ^ 1) * HWORDS, HWORDS)], hist2)

    cls0 = lane * 16
    cls1 = (lane + 16) * 16
    cnt0 = zeros
    cnt1 = zeros
    for l in range(16):
        cnt0 = cnt0 + plsc.load_gather(hist, [cls0 + l])
        cnt0 = cnt0 + plsc.load_gather(hist2, [cls0 + l])
        cnt1 = cnt1 + plsc.load_gather(hist, [cls1 + l])
        cnt1 = cnt1 + plsc.load_gather(hist2, [cls1 + l])

    inv0 = jnp.where(cnt0 > 0.0, ones / jnp.maximum(cnt0, 1.0), zeros)
    inv1 = jnp.where(cnt1 > 0.0, ones / jnp.maximum(cnt1, 1.0), zeros)
    denom = lax.broadcast(jnp.sum(inv0 + inv1), (16,))
    wtab[pl.ds(0, 16)] = inv0 / denom
    wtab[pl.ds(16, 16)] = inv1 / denom

    half = jnp.full((16,), 0x8000, jnp.int32)
    himask = jnp.full((16,), -65536, jnp.int32)
    m31 = jnp.full((16,), 31, jnp.int32)
    for j in range(64):
        keys = lane + (16 * j)
        ka = keys & m31
        kb = jax.lax.shift_right_logical(keys, 5)
        wa = plsc.bitcast(plsc.load_gather(wtab, [ka]), jnp.int32)
        wb = plsc.bitcast(plsc.load_gather(wtab, [kb]), jnp.int32)
        lo = jax.lax.shift_right_logical(wa + half, 16)
        hi = (wb + half) & himask
        ptab[pl.ds(16 * j, 16)] = lo | hi

    m1023 = jnp.full((16,), 1023, jnp.int32)
    wrs = [None, None]
    for ci in range(NCH):
        k = ci % 2
        if wrs[k] is not None:
            wrs[k].wait()
        obuf = bufs[k]
        pbase = ci * (CR * 32)

        @plsc.parallel_loop(0, CR, step=1, unroll=2)
        def _g(r):
            for u in range(2):
                packed = pbuf[pl.ds(pbase + r * 32 + u * 16, 16)]
                k01 = packed & m1023
                k23 = jax.lax.shift_right_logical(packed, 10)
                g01 = plsc.load_gather(ptab, [k01])
                g23 = plsc.load_gather(ptab, [k23])
                obuf[r, pl.ds(u * 64, 16)] = g01 << 16
                obuf[r, pl.ds(u * 64 + 16, 16)] = g01 & himask
                obuf[r, pl.ds(u * 64 + 32, 16)] = g23 << 16
                obuf[r, pl.ds(u * 64 + 48, 16)] = g23 & himask

        wrs[k] = pltpu.async_copy(
            obuf, out_hbm.at[pl.ds(row0(ci), CR), :], semos[k])

    wrs[0].wait()
    wrs[1].wait()


def kernel(inputs):
    mask = inputs.astype(jnp.int32).reshape(ROWS, 128)
    out = _sc_weight_map(mask)
    return jax.lax.bitcast_convert_type(out, jnp.float32).reshape(
        NB, 512, 512, 1)


# speedup vs baseline: 1.2290x; 1.2290x over previous
import functools

import jax
import jax.numpy as jnp
from jax import lax
from jax.experimental import pallas as pl
from jax.experimental.pallas import tpu as pltpu
from jax.experimental.pallas import tpu_sc as plsc

NCLS = 21
NB = 16
NPIX = 512 * 512
ROWS = NB * NPIX // 128
SROWS = NPIX // 128
WROWS = SROWS // 2
CR = 128
NCH = WROWS // CR
HWORDS = 512
PWORDS = WROWS * 32

_mesh = plsc.VectorSubcoreMesh(core_axis_name="c", subcore_axis_name="s")


@functools.partial(
    pl.kernel,
    out_type=jax.ShapeDtypeStruct((ROWS, 128), jnp.float32),
    mesh=_mesh,
    compiler_params=pltpu.CompilerParams(needs_layout_passes=False),
    scratch_types=[
        pltpu.VMEM((CR, 128), jnp.int32),
        pltpu.VMEM((CR, 128), jnp.int32),
        pltpu.VMEM((CR, 128), jnp.float32),
        pltpu.VMEM((CR, 128), jnp.float32),
        pltpu.VMEM((PWORDS,), jnp.int32),
        pltpu.VMEM((HWORDS,), jnp.float32),
        pltpu.VMEM((HWORDS,), jnp.float32),
        pltpu.VMEM((32,), jnp.float32),
        pltpu.VMEM((1024,), jnp.int32),
        pltpu.VMEM_SHARED((16 * HWORDS,), jnp.float32),
        pltpu.SemaphoreType.DMA,
        pltpu.SemaphoreType.DMA,
        pltpu.SemaphoreType.DMA,
        pltpu.SemaphoreType.DMA,
    ],
)
def _sc_weight_map(mask_hbm, out_hbm, buf0, buf1, obuf0, obuf1,
                   pbuf, hist, hist2, wtab, ptab, shist,
                   semi0, semi1, semo0, semo1):
    c = lax.axis_index("c")
    s = lax.axis_index("s")
    b = c * 8 + s // 2
    rbase = b * SROWS + (s % 2) * WROWS

    lane = lax.iota(jnp.int32, 16)
    zeros = jnp.zeros((16,), jnp.float32)
    ones = jnp.ones((16,), jnp.float32)

    bufs = [buf0, buf1]
    obufs = [obuf0, obuf1]
    semis = [semi0, semi1]
    semos = [semo0, semo1]

    def row0(ci):
        return rbase + ci * CR

    for k in range(32):
        hist[pl.ds(16 * k, 16)] = zeros

    cps = [
        pltpu.async_copy(mask_hbm.at[pl.ds(row0(0), CR), :], buf0, semi0),
        pltpu.async_copy(mask_hbm.at[pl.ds(row0(1), CR), :], buf1, semi1),
    ]
    for ci in range(NCH):
        k = ci % 2
        cps[k].wait()
        buf = bufs[k]
        pbase = ci * (CR * 32)

        @plsc.parallel_loop(0, CR, step=1, unroll=2)
        def _h(r):
            for u in range(2):
                vs = []
                for t in range(4):
                    v = buf[r, pl.ds(u * 64 + t * 16, 16)]
                    plsc.addupdate_scatter(hist, [v * 16 + lane], ones)
                    vs.append(v)
                packed = (vs[0] | (vs[1] << 5)) | ((vs[2] << 10) | (vs[3] << 15))
                pbuf[pl.ds(pbase + r * 32 + u * 16, 16)] = packed

        if ci + 2 < NCH:
            cps[k] = pltpu.async_copy(
                mask_hbm.at[pl.ds(row0(ci + 2), CR), :], bufs[k], semis[k])

    pltpu.sync_copy(hist, shist.at[pl.ds(s * HWORDS, HWORDS)])
    plsc.subcore_barrier()
    pltpu.sync_copy(shist.at[pl.ds((s ^ 1) * HWORDS, HWORDS)], hist2)

    cls0 = lane * 16
    cls1 = (lane + 16) * 16
    cnt0 = zeros
    cnt1 = zeros
    for l in range(16):
        cnt0 = cnt0 + plsc.load_gather(hist, [cls0 + l])
        cnt0 = cnt0 + plsc.load_gather(hist2, [cls0 + l])
        cnt1 = cnt1 + plsc.load_gather(hist, [cls1 + l])
        cnt1 = cnt1 + plsc.load_gather(hist2, [cls1 + l])

    inv0 = jnp.where(cnt0 > 0.0, ones / jnp.maximum(cnt0, 1.0), zeros)
    inv1 = jnp.where(cnt1 > 0.0, ones / jnp.maximum(cnt1, 1.0), zeros)
    denom = lax.broadcast(jnp.sum(inv0 + inv1), (16,))
    wtab[pl.ds(0, 16)] = inv0 / denom
    wtab[pl.ds(16, 16)] = inv1 / denom

    half = jnp.full((16,), 0x8000, jnp.int32)
    himask = jnp.full((16,), -65536, jnp.int32)
    m31 = jnp.full((16,), 31, jnp.int32)
    for j in range(64):
        keys = lane + (16 * j)
        ka = keys & m31
        kb = jax.lax.shift_right_logical(keys, 5)
        wa = plsc.bitcast(plsc.load_gather(wtab, [ka]), jnp.int32)
        wb = plsc.bitcast(plsc.load_gather(wtab, [kb]), jnp.int32)
        lo = jax.lax.shift_right_logical(wa + half, 16)
        hi = (wb + half) & himask
        ptab[pl.ds(16 * j, 16)] = lo | hi

    m1023 = jnp.full((16,), 1023, jnp.int32)
    wrs = [None, None]
    for ci in range(NCH):
        k = ci % 2
        if wrs[k] is not None:
            wrs[k].wait()
        obuf = obufs[k]
        pbase = ci * (CR * 32)

        @plsc.parallel_loop(0, CR, step=1, unroll=2)
        def _g(r):
            for u in range(2):
                packed = pbuf[pl.ds(pbase + r * 32 + u * 16, 16)]
                k01 = packed & m1023
                k23 = jax.lax.shift_right_logical(packed, 10)
                g01 = plsc.load_gather(ptab, [k01])
                g23 = plsc.load_gather(ptab, [k23])
                obuf[r, pl.ds(u * 64, 16)] = plsc.bitcast(
                    g01 << 16, jnp.float32)
                obuf[r, pl.ds(u * 64 + 16, 16)] = plsc.bitcast(
                    g01 & himask, jnp.float32)
                obuf[r, pl.ds(u * 64 + 32, 16)] = plsc.bitcast(
                    g23 << 16, jnp.float32)
                obuf[r, pl.ds(u * 64 + 48, 16)] = plsc.bitcast(
                    g23 & himask, jnp.float32)

        wrs[k] = pltpu.async_copy(
            obuf, out_hbm.at[pl.ds(row0(ci), CR), :], semos[k])

    wrs[0].wait()
    wrs[1].wait()


def kernel(inputs):
    mask = inputs.astype(jnp.int32).reshape(ROWS, 128)
    out = _sc_weight_map(mask)
    return out.reshape(NB, 512, 512, 1)


# speedup vs baseline: 1.2599x; 1.0252x over previous
import functools

import jax
import jax.numpy as jnp
from jax import lax
from jax.experimental import pallas as pl
from jax.experimental.pallas import tpu as pltpu
from jax.experimental.pallas import tpu_sc as plsc

NCLS = 21
NB = 16
NPIX = 512 * 512
ROWS = NB * NPIX // 128
SROWS = NPIX // 128
WROWS = SROWS // 2
CR = 128
NCH = WROWS // CR
HWORDS = 512
PWORDS = WROWS * 32

_mesh = plsc.VectorSubcoreMesh(core_axis_name="c", subcore_axis_name="s")


@functools.partial(
    pl.kernel,
    out_type=jax.ShapeDtypeStruct((ROWS, 128), jnp.float32),
    mesh=_mesh,
    compiler_params=pltpu.CompilerParams(needs_layout_passes=False),
    scratch_types=[
        pltpu.VMEM((CR, 128), jnp.int32),
        pltpu.VMEM((CR, 128), jnp.int32),
        pltpu.VMEM((CR, 128), jnp.float32),
        pltpu.VMEM((CR, 128), jnp.float32),
        pltpu.VMEM((PWORDS,), jnp.int32),
        pltpu.VMEM((HWORDS,), jnp.float32),
        pltpu.VMEM((HWORDS,), jnp.float32),
        pltpu.VMEM((32,), jnp.float32),
        pltpu.VMEM_SHARED((16 * HWORDS,), jnp.float32),
        pltpu.SemaphoreType.DMA,
        pltpu.SemaphoreType.DMA,
        pltpu.SemaphoreType.DMA,
        pltpu.SemaphoreType.DMA,
    ],
)
def _sc_weight_map(mask_hbm, out_hbm, buf0, buf1, obuf0, obuf1,
                   pbuf, hist, hist2, wtab, shist,
                   semi0, semi1, semo0, semo1):
    c = lax.axis_index("c")
    s = lax.axis_index("s")
    b = c * 8 + s // 2
    rbase = b * SROWS + (s % 2) * WROWS

    lane = lax.iota(jnp.int32, 16)
    zeros = jnp.zeros((16,), jnp.float32)
    ones = jnp.ones((16,), jnp.float32)

    bufs = [buf0, buf1]
    obufs = [obuf0, obuf1]
    semis = [semi0, semi1]
    semos = [semo0, semo1]

    def row0(ci):
        return rbase + ci * CR

    for k in range(32):
        hist[pl.ds(16 * k, 16)] = zeros

    cps = [
        pltpu.async_copy(mask_hbm.at[pl.ds(row0(0), CR), :], buf0, semi0),
        pltpu.async_copy(mask_hbm.at[pl.ds(row0(1), CR), :], buf1, semi1),
    ]
    for ci in range(NCH):
        k = ci % 2
        cps[k].wait()
        buf = bufs[k]
        pbase = ci * (CR * 32)

        @plsc.parallel_loop(0, CR, step=1, unroll=1)
        def _h(r):
            for u in range(2):
                vs = []
                for t in range(4):
                    v = buf[r, pl.ds(u * 64 + t * 16, 16)]
                    plsc.addupdate_scatter(hist, [v * 16 + lane], ones)
                    vs.append(v)
                packed = (vs[0] | (vs[1] << 8)) | ((vs[2] << 16) | (vs[3] << 24))
                pbuf[pl.ds(pbase + r * 32 + u * 16, 16)] = packed

        if ci + 2 < NCH:
            cps[k] = pltpu.async_copy(
                mask_hbm.at[pl.ds(row0(ci + 2), CR), :], bufs[k], semis[k])

    pltpu.sync_copy(hist, shist.at[pl.ds(s * HWORDS, HWORDS)])
    plsc.subcore_barrier()
    pltpu.sync_copy(shist.at[pl.ds((s ^ 1) * HWORDS, HWORDS)], hist2)

    cls0 = lane * 16
    cls1 = (lane + 16) * 16
    cnt0 = zeros
    cnt1 = zeros
    for l in range(16):
        cnt0 = cnt0 + plsc.load_gather(hist, [cls0 + l])
        cnt0 = cnt0 + plsc.load_gather(hist2, [cls0 + l])
        cnt1 = cnt1 + plsc.load_gather(hist, [cls1 + l])
        cnt1 = cnt1 + plsc.load_gather(hist2, [cls1 + l])

    inv0 = jnp.where(cnt0 > 0.0, ones / jnp.maximum(cnt0, 1.0), zeros)
    inv1 = jnp.where(cnt1 > 0.0, ones / jnp.maximum(cnt1, 1.0), zeros)
    denom = lax.broadcast(jnp.sum(inv0 + inv1), (16,))
    wtab[pl.ds(0, 16)] = inv0 / denom
    wtab[pl.ds(16, 16)] = inv1 / denom

    mask255 = jnp.full((16,), 255, jnp.int32)
    wrs = [None, None]
    for ci in range(NCH):
        k = ci % 2
        if wrs[k] is not None:
            wrs[k].wait()
        obuf = obufs[k]
        pbase = ci * (CR * 32)

        @plsc.parallel_loop(0, CR, step=1, unroll=1)
        def _g(r):
            for u in range(2):
                packed = pbuf[pl.ds(pbase + r * 32 + u * 16, 16)]
                b0 = packed & mask255
                b1 = (packed >> 8) & mask255
                b2 = (packed >> 16) & mask255
                b3 = packed >> 24
                for t, vv in enumerate((b0, b1, b2, b3)):
                    obuf[r, pl.ds(u * 64 + t * 16, 16)] = (
                        plsc.load_gather(wtab, [vv]))

        wrs[k] = pltpu.async_copy(
            obuf, out_hbm.at[pl.ds(row0(ci), CR), :], semos[k])

    wrs[0].wait()
    wrs[1].wait()


def kernel(inputs):
    mask = inputs.astype(jnp.int32).reshape(ROWS, 128)
    out = _sc_weight_map(mask)
    return out.reshape(NB, 512, 512, 1)


# speedup vs baseline: 1.4139x; 1.1222x over previous
import functools

import jax
import jax.numpy as jnp
from jax import lax
from jax.experimental import pallas as pl
from jax.experimental.pallas import tpu as pltpu
from jax.experimental.pallas import tpu_sc as plsc

NCLS = 21
NB = 16
NPIX = 512 * 512
ROWS = NB * NPIX // 128
SROWS = NPIX // 128
WROWS = SROWS // 2
CR = 128
NCH = WROWS // CR
HWORDS = 512
PWORDS = WROWS * 32

_mesh = plsc.VectorSubcoreMesh(core_axis_name="c", subcore_axis_name="s")


@functools.partial(
    pl.kernel,
    out_type=jax.ShapeDtypeStruct((ROWS, 128), jnp.float32),
    mesh=_mesh,
    compiler_params=pltpu.CompilerParams(needs_layout_passes=False),
    scratch_types=[
        pltpu.VMEM((CR, 128), jnp.int32),
        pltpu.VMEM((CR, 128), jnp.int32),
        pltpu.VMEM((CR, 128), jnp.float32),
        pltpu.VMEM((CR, 128), jnp.float32),
        pltpu.VMEM((PWORDS,), jnp.int32),
        pltpu.VMEM((HWORDS,), jnp.float32),
        pltpu.VMEM((HWORDS,), jnp.float32),
        pltpu.VMEM((32,), jnp.float32),
        pltpu.VMEM_SHARED((16 * HWORDS,), jnp.float32),
        pltpu.SemaphoreType.DMA,
        pltpu.SemaphoreType.DMA,
        pltpu.SemaphoreType.DMA,
        pltpu.SemaphoreType.DMA,
    ],
)
def _sc_weight_map(mask_hbm, out_hbm, buf0, buf1, obuf0, obuf1,
                   pbuf, hist, hist2, wtab, shist,
                   semi0, semi1, semo0, semo1):
    c = lax.axis_index("c")
    s = lax.axis_index("s")
    b = c * 8 + s // 2
    rbase = b * SROWS + (s % 2) * WROWS

    lane = lax.iota(jnp.int32, 16)
    zeros = jnp.zeros((16,), jnp.float32)
    ones = jnp.ones((16,), jnp.float32)

    bufs = [buf0, buf1]
    obufs = [obuf0, obuf1]
    semis = [semi0, semi1]
    semos = [semo0, semo1]

    def row0(ci):
        return rbase + ci * CR

    for k in range(32):
        hist[pl.ds(16 * k, 16)] = zeros

    cps = [
        pltpu.async_copy(mask_hbm.at[pl.ds(row0(0), CR), :], buf0, semi0),
        pltpu.async_copy(mask_hbm.at[pl.ds(row0(1), CR), :], buf1, semi1),
    ]
    def _p1_pair(j, carry):
        for k in range(2):
            ci = j * 2 + k
            pltpu.make_async_copy(
                mask_hbm.at[pl.ds(row0(0), CR), :], bufs[k], semis[k]).wait()
            buf = bufs[k]
            pbase = ci * (CR * 32)

            @plsc.parallel_loop(0, CR, step=1, unroll=1)
            def _h(r):
                for u in range(2):
                    vs = []
                    for t in range(4):
                        v = buf[r, pl.ds(u * 64 + t * 16, 16)]
                        plsc.addupdate_scatter(hist, [v * 16 + lane], ones)
                        vs.append(v)
                    packed = (vs[0] | (vs[1] << 8)) | (
                        (vs[2] << 16) | (vs[3] << 24))
                    pbuf[pl.ds(pbase + r * 32 + u * 16, 16)] = packed

            @pl.when(j < NCH // 2 - 1)
            def _():
                pltpu.async_copy(
                    mask_hbm.at[pl.ds(row0(ci + 2), CR), :],
                    bufs[k], semis[k])

        return carry

    lax.fori_loop(0, NCH // 2, _p1_pair, 0)

    pltpu.sync_copy(hist, shist.at[pl.ds(s * HWORDS, HWORDS)])
    plsc.subcore_barrier()
    pltpu.sync_copy(shist.at[pl.ds((s ^ 1) * HWORDS, HWORDS)], hist2)

    cls0 = lane * 16
    cls1 = (lane + 16) * 16
    cnt0 = zeros
    cnt1 = zeros
    for l in range(16):
        cnt0 = cnt0 + plsc.load_gather(hist, [cls0 + l])
        cnt0 = cnt0 + plsc.load_gather(hist2, [cls0 + l])
        cnt1 = cnt1 + plsc.load_gather(hist, [cls1 + l])
        cnt1 = cnt1 + plsc.load_gather(hist2, [cls1 + l])

    inv0 = jnp.where(cnt0 > 0.0, ones / jnp.maximum(cnt0, 1.0), zeros)
    inv1 = jnp.where(cnt1 > 0.0, ones / jnp.maximum(cnt1, 1.0), zeros)
    denom = lax.broadcast(jnp.sum(inv0 + inv1), (16,))
    wtab[pl.ds(0, 16)] = inv0 / denom
    wtab[pl.ds(16, 16)] = inv1 / denom

    mask255 = jnp.full((16,), 255, jnp.int32)

    def _p2_pair(j, carry):
        for k in range(2):
            ci = j * 2 + k
            obuf = obufs[k]

            @pl.when(j > 0)
            def _():
                pltpu.make_async_copy(
                    obuf, out_hbm.at[pl.ds(row0(0), CR), :], semos[k]).wait()

            pbase = ci * (CR * 32)

            @plsc.parallel_loop(0, CR, step=1, unroll=1)
            def _g(r):
                for u in range(2):
                    packed = pbuf[pl.ds(pbase + r * 32 + u * 16, 16)]
                    b0 = packed & mask255
                    b1 = (packed >> 8) & mask255
                    b2 = (packed >> 16) & mask255
                    b3 = packed >> 24
                    for t, vv in enumerate((b0, b1, b2, b3)):
                        obuf[r, pl.ds(u * 64 + t * 16, 16)] = (
                            plsc.load_gather(wtab, [vv]))

            pltpu.async_copy(
                obuf, out_hbm.at[pl.ds(row0(ci), CR), :], semos[k])

        return carry

    lax.fori_loop(0, NCH // 2, _p2_pair, 0)
    for k in range(2):
        pltpu.make_async_copy(
            obufs[k], out_hbm.at[pl.ds(row0(0), CR), :], semos[k]).wait()


def kernel(inputs):
    mask = inputs.astype(jnp.int32).reshape(ROWS, 128)
    out = _sc_weight_map(mask)
    return out.reshape(NB, 512, 512, 1)


# speedup vs baseline: 1.4140x; 1.0001x over previous
import functools

import jax
import jax.numpy as jnp
from jax import lax
from jax.experimental import pallas as pl
from jax.experimental.pallas import tpu as pltpu
from jax.experimental.pallas import tpu_sc as plsc

NCLS = 21
NB = 16
NPIX = 512 * 512
ROWS = NB * NPIX // 128
SROWS = NPIX // 128
WROWS = SROWS // 2
CR = 128
NCH = WROWS // CR
HWORDS = 512
PWORDS = WROWS * 32

_mesh = plsc.VectorSubcoreMesh(core_axis_name="c", subcore_axis_name="s")


@functools.partial(
    pl.kernel,
    out_type=jax.ShapeDtypeStruct((ROWS, 128), jnp.float32),
    mesh=_mesh,
    compiler_params=pltpu.CompilerParams(needs_layout_passes=False),
    scratch_types=[
        pltpu.VMEM((CR, 128), jnp.int32),
        pltpu.VMEM((CR, 128), jnp.int32),
        pltpu.VMEM((CR, 128), jnp.float32),
        pltpu.VMEM((CR, 128), jnp.float32),
        pltpu.VMEM((PWORDS,), jnp.int32),
        pltpu.VMEM((HWORDS,), jnp.float32),
        pltpu.VMEM((HWORDS,), jnp.float32),
        pltpu.VMEM((32,), jnp.float32),
        pltpu.VMEM_SHARED((16 * HWORDS,), jnp.float32),
        pltpu.SemaphoreType.DMA,
        pltpu.SemaphoreType.DMA,
        pltpu.SemaphoreType.DMA,
        pltpu.SemaphoreType.DMA,
    ],
)
def _sc_weight_map(mask_hbm, out_hbm, buf0, buf1, obuf0, obuf1,
                   pbuf, hist, hist2, wtab, shist,
                   semi0, semi1, semo0, semo1):
    c = lax.axis_index("c")
    s = lax.axis_index("s")
    b = c * 8 + s // 2
    rbase = b * SROWS + (s % 2) * WROWS

    lane = lax.iota(jnp.int32, 16)
    zeros = jnp.zeros((16,), jnp.float32)
    ones = jnp.ones((16,), jnp.float32)

    bufs = [buf0, buf1]
    obufs = [obuf0, obuf1]
    semis = [semi0, semi1]
    semos = [semo0, semo1]

    def row0(ci):
        return rbase + ci * CR

    def _z(k, carry):
        hist[pl.ds(16 * k, 16)] = zeros
        return carry

    lax.fori_loop(0, 32, _z, 0)

    cps = [
        pltpu.async_copy(mask_hbm.at[pl.ds(row0(0), CR), :], buf0, semi0),
        pltpu.async_copy(mask_hbm.at[pl.ds(row0(1), CR), :], buf1, semi1),
    ]
    def _p1_pair(j, carry):
        for k in range(2):
            ci = j * 2 + k
            pltpu.make_async_copy(
                mask_hbm.at[pl.ds(row0(0), CR), :], bufs[k], semis[k]).wait()
            buf = bufs[k]
            pbase = ci * (CR * 32)

            @plsc.parallel_loop(0, CR, step=1, unroll=1)
            def _h(r):
                for u in range(2):
                    vs = []
                    for t in range(4):
                        v = buf[r, pl.ds(u * 64 + t * 16, 16)]
                        plsc.addupdate_scatter(hist, [v * 16 + lane], ones)
                        vs.append(v)
                    packed = (vs[0] | (vs[1] << 8)) | (
                        (vs[2] << 16) | (vs[3] << 24))
                    pbuf[pl.ds(pbase + r * 32 + u * 16, 16)] = packed

            @pl.when(j < NCH // 2 - 1)
            def _():
                pltpu.async_copy(
                    mask_hbm.at[pl.ds(row0(ci + 2), CR), :],
                    bufs[k], semis[k])

        return carry

    lax.fori_loop(0, NCH // 2, _p1_pair, 0)

    pltpu.sync_copy(hist, shist.at[pl.ds(s * HWORDS, HWORDS)])
    plsc.subcore_barrier()
    pltpu.sync_copy(shist.at[pl.ds((s ^ 1) * HWORDS, HWORDS)], hist2)

    cls0 = lane * 16
    cls1 = (lane + 16) * 16
    def _red(l, carry):
        c0, c1 = carry
        c0 = c0 + plsc.load_gather(hist, [cls0 + l])
        c0 = c0 + plsc.load_gather(hist2, [cls0 + l])
        c1 = c1 + plsc.load_gather(hist, [cls1 + l])
        c1 = c1 + plsc.load_gather(hist2, [cls1 + l])
        return (c0, c1)

    cnt0, cnt1 = lax.fori_loop(0, 16, _red, (zeros, zeros))

    inv0 = jnp.where(cnt0 > 0.0, ones / jnp.maximum(cnt0, 1.0), zeros)
    inv1 = jnp.where(cnt1 > 0.0, ones / jnp.maximum(cnt1, 1.0), zeros)
    denom = lax.broadcast(jnp.sum(inv0 + inv1), (16,))
    wtab[pl.ds(0, 16)] = inv0 / denom
    wtab[pl.ds(16, 16)] = inv1 / denom

    mask255 = jnp.full((16,), 255, jnp.int32)

    def _p2_pair(j, carry):
        for k in range(2):
            ci = j * 2 + k
            obuf = obufs[k]

            @pl.when(j > 0)
            def _():
                pltpu.make_async_copy(
                    obuf, out_hbm.at[pl.ds(row0(0), CR), :], semos[k]).wait()

            pbase = ci * (CR * 32)

            @plsc.parallel_loop(0, CR, step=1, unroll=1)
            def _g(r):
                for u in range(2):
                    packed = pbuf[pl.ds(pbase + r * 32 + u * 16, 16)]
                    b0 = packed & mask255
                    b1 = (packed >> 8) & mask255
                    b2 = (packed >> 16) & mask255
                    b3 = packed >> 24
                    for t, vv in enumerate((b0, b1, b2, b3)):
                        obuf[r, pl.ds(u * 64 + t * 16, 16)] = (
                            plsc.load_gather(wtab, [vv]))

            pltpu.async_copy(
                obuf, out_hbm.at[pl.ds(row0(ci), CR), :], semos[k])

        return carry

    lax.fori_loop(0, NCH // 2, _p2_pair, 0)
    for k in range(2):
        pltpu.make_async_copy(
            obufs[k], out_hbm.at[pl.ds(row0(0), CR), :], semos[k]).wait()


def kernel(inputs):
    mask = inputs.astype(jnp.int32).reshape(ROWS, 128)
    out = _sc_weight_map(mask)
    return out.reshape(NB, 512, 512, 1)
